# baseline (device time: 25522 ns/iter reference)
import functools

import jax
import jax.numpy as jnp
from jax import lax
from jax.experimental import pallas as pl
from jax.experimental.pallas import tpu as pltpu

N_DEV = 4
H_GLOBAL = 512


def kernel(x, Wp):
    b, h_per, w, c = x.shape
    c_out = Wp.shape[1]
    n_norm = float(H_GLOBAL * w)

    xt = jnp.transpose(x, (0, 1, 3, 2))

    def body(x_ref, wp_ref, out_ref, local_ref, stats_ref, send_sems, recv_sems):
        my = lax.axis_index("i")

        barrier_sem = pltpu.get_barrier_semaphore()
        for off in (1, 2, 3):
            pl.semaphore_signal(
                barrier_sem, inc=1,
                device_id=((my + off) % N_DEV,),
                device_id_type=pl.DeviceIdType.MESH,
            )
        pl.semaphore_wait(barrier_sem, N_DEV - 1)

        xv = x_ref[...]
        ps = jnp.sum(xv, axis=(1, 3))
        pss = jnp.sum(xv * xv, axis=(1, 3))
        local_ref[...] = jnp.concatenate([ps, pss], axis=0)

        rdmas = []
        for off in (1, 2, 3):
            rdma = pltpu.make_async_remote_copy(
                src_ref=local_ref,
                dst_ref=stats_ref.at[off - 1],
                send_sem=send_sems.at[off - 1],
                recv_sem=recv_sems.at[off - 1],
                device_id=((my + off) % N_DEV,),
                device_id_type=pl.DeviceIdType.MESH,
            )
            rdma.start()
            rdmas.append(rdma)
        for rdma in rdmas:
            rdma.wait_recv()

        tot = local_ref[...] + stats_ref[0] + stats_ref[1] + stats_ref[2]
        mean = tot[:b, :] / n_norm
        var = tot[b:, :] / n_norm - mean * mean
        inv = lax.rsqrt(var + 1e-5)

        hn = (xv - mean[:, None, :, None]) * inv[:, None, :, None]
        a = hn * jax.nn.sigmoid(hn)
        a2 = a.astype(jnp.bfloat16)
        wb = wp_ref[...].astype(jnp.bfloat16)
        o = lax.dot_general(
            a2, wb,
            dimension_numbers=(((2,), (0,)), ((), ())),
            preferred_element_type=jnp.float32,
        )
        out_ref[...] = o.astype(jnp.bfloat16)

        for rdma in rdmas:
            rdma.wait_send()

        @functools.partial(
            pl.run_scoped, exit_sem=pltpu.SemaphoreType.REGULAR
        )
        def _(exit_sem):
            for off in (1, 2, 3):
                pl.semaphore_signal(
                    exit_sem, inc=1,
                    device_id=((my + off) % N_DEV,),
                    device_id_type=pl.DeviceIdType.MESH,
                )
            pl.semaphore_wait(exit_sem, N_DEV - 1)

    return pl.pallas_call(
        body,
        out_shape=jax.ShapeDtypeStruct((b, h_per, w, c_out), jnp.bfloat16),
        in_specs=[
            pl.BlockSpec(memory_space=pltpu.VMEM),
            pl.BlockSpec(memory_space=pltpu.VMEM),
        ],
        out_specs=pl.BlockSpec(memory_space=pltpu.VMEM),
        scratch_shapes=[
            pltpu.VMEM((2 * b, c), jnp.float32),
            pltpu.VMEM((N_DEV - 1, 2 * b, c), jnp.float32),
            pltpu.SemaphoreType.DMA((N_DEV - 1,)),
            pltpu.SemaphoreType.DMA((N_DEV - 1,)),
        ],
        compiler_params=pltpu.CompilerParams(
            collective_id=0,
            vmem_limit_bytes=60 * 1024 * 1024,
        ),
    )(xt, Wp)


# device time: 18525 ns/iter; 1.3777x vs baseline; 1.3777x over previous
import functools

import jax
import jax.numpy as jnp
from jax import lax
from jax.experimental import pallas as pl
from jax.experimental.pallas import tpu as pltpu

N_DEV = 4
H_GLOBAL = 512


def kernel(x, Wp):
    b, h_per, w, c = x.shape
    c_out = Wp.shape[1]
    n_norm = float(H_GLOBAL * w)

    xt = jnp.transpose(x, (0, 1, 3, 2))

    def body(x_ref, wp_ref, out_ref, local_ref, stats_ref, send_sems, recv_sems):
        my = lax.axis_index("i")

        barrier_sem = pltpu.get_barrier_semaphore()
        for off in (1, 2, 3):
            pl.semaphore_signal(
                barrier_sem, inc=1,
                device_id=((my + off) % N_DEV,),
                device_id_type=pl.DeviceIdType.MESH,
            )
        pl.semaphore_wait(barrier_sem, N_DEV - 1)

        xv = x_ref[...]
        ps = jnp.sum(xv, axis=(1, 3))
        pss = jnp.sum(xv * xv, axis=(1, 3))
        local_ref[...] = jnp.concatenate([ps, pss], axis=0)

        rdmas = []
        for off in (1, 2, 3):
            rdma = pltpu.make_async_remote_copy(
                src_ref=local_ref,
                dst_ref=stats_ref.at[off - 1],
                send_sem=send_sems.at[off - 1],
                recv_sem=recv_sems.at[off - 1],
                device_id=((my + off) % N_DEV,),
                device_id_type=pl.DeviceIdType.MESH,
            )
            rdma.start()
            rdmas.append(rdma)
        for rdma in rdmas:
            rdma.wait_recv()

        tot = local_ref[...] + stats_ref[0] + stats_ref[1] + stats_ref[2]
        mean = tot[:b, :] / n_norm
        var = tot[b:, :] / n_norm - mean * mean
        inv = lax.rsqrt(var + 1e-5)

        mb = mean.astype(jnp.bfloat16)[:, None, :, None]
        ib = inv.astype(jnp.bfloat16)[:, None, :, None]
        hn = (xv.astype(jnp.bfloat16) - mb) * ib
        a2 = hn * jax.nn.sigmoid(hn)
        wb = wp_ref[...].astype(jnp.bfloat16)
        o = lax.dot_general(
            a2, wb,
            dimension_numbers=(((2,), (0,)), ((), ())),
            preferred_element_type=jnp.float32,
        )
        out_ref[...] = o.astype(jnp.bfloat16)

        for rdma in rdmas:
            rdma.wait_send()

        @functools.partial(
            pl.run_scoped, exit_sem=pltpu.SemaphoreType.REGULAR
        )
        def _(exit_sem):
            for off in (1, 2, 3):
                pl.semaphore_signal(
                    exit_sem, inc=1,
                    device_id=((my + off) % N_DEV,),
                    device_id_type=pl.DeviceIdType.MESH,
                )
            pl.semaphore_wait(exit_sem, N_DEV - 1)

    return pl.pallas_call(
        body,
        out_shape=jax.ShapeDtypeStruct((b, h_per, w, c_out), jnp.bfloat16),
        in_specs=[
            pl.BlockSpec(memory_space=pltpu.VMEM),
            pl.BlockSpec(memory_space=pltpu.VMEM),
        ],
        out_specs=pl.BlockSpec(memory_space=pltpu.VMEM),
        scratch_shapes=[
            pltpu.VMEM((2 * b, c), jnp.float32),
            pltpu.VMEM((N_DEV - 1, 2 * b, c), jnp.float32),
            pltpu.SemaphoreType.DMA((N_DEV - 1,)),
            pltpu.SemaphoreType.DMA((N_DEV - 1,)),
        ],
        compiler_params=pltpu.CompilerParams(collective_id=0),
    )(xt, Wp)


# device time: 16134 ns/iter; 1.5819x vs baseline; 1.1482x over previous
import jax
import jax.numpy as jnp
from jax import lax
from jax.experimental import pallas as pl
from jax.experimental.pallas import tpu as pltpu

N_DEV = 4
H_GLOBAL = 512


def kernel(x, Wp):
    b, h_per, w, c = x.shape
    c_out = Wp.shape[1]
    n_norm = float(H_GLOBAL * w)

    xt = jnp.transpose(x, (0, 1, 3, 2))

    def body(x_ref, wp_ref, out_ref, local_ref, stats_ref, tot_ref,
             send_sems, recv_sems, exit_sem):
        my = lax.axis_index("i")

        barrier_sem = pltpu.get_barrier_semaphore()
        for off in (1, 2, 3):
            pl.semaphore_signal(
                barrier_sem, inc=1,
                device_id=((my + off) % N_DEV,),
                device_id_type=pl.DeviceIdType.MESH,
            )

        xv = x_ref[...]
        ps = jnp.sum(xv, axis=(1, 3))
        pss = jnp.sum(xv * xv, axis=(1, 3))
        local_ref[...] = jnp.concatenate([ps, pss], axis=0)

        pl.semaphore_wait(barrier_sem, N_DEV - 1)

        rdmas = []
        for off in (1, 2, 3):
            rdma = pltpu.make_async_remote_copy(
                src_ref=local_ref,
                dst_ref=stats_ref.at[off - 1],
                send_sem=send_sems.at[off - 1],
                recv_sem=recv_sems.at[off - 1],
                device_id=((my + off) % N_DEV,),
                device_id_type=pl.DeviceIdType.MESH,
            )
            rdma.start()
            rdmas.append(rdma)
        for rdma in rdmas:
            rdma.wait_recv()
        for rdma in rdmas:
            rdma.wait_send()

        tot_ref[...] = (
            local_ref[...] + stats_ref[0] + stats_ref[1] + stats_ref[2]
        )

        for off in (1, 2, 3):
            pl.semaphore_signal(
                exit_sem, inc=1,
                device_id=((my + off) % N_DEV,),
                device_id_type=pl.DeviceIdType.MESH,
            )

        tot = tot_ref[...]
        mean = tot[:b, :] / n_norm
        var = tot[b:, :] / n_norm - mean * mean
        inv = lax.rsqrt(var + 1e-5)
        mb = mean.astype(jnp.bfloat16)[:, None, :, None]
        ib = inv.astype(jnp.bfloat16)[:, None, :, None]

        hn = (xv.astype(jnp.bfloat16) - mb) * ib
        a2 = hn * jax.nn.sigmoid(hn)
        wb = wp_ref[...].astype(jnp.bfloat16)
        o = lax.dot_general(
            a2, wb,
            dimension_numbers=(((2,), (0,)), ((), ())),
            preferred_element_type=jnp.float32,
        )
        out_ref[...] = o.astype(jnp.bfloat16)

        pl.semaphore_wait(exit_sem, N_DEV - 1)

    return pl.pallas_call(
        body,
        out_shape=jax.ShapeDtypeStruct((b, h_per, w, c_out), jnp.bfloat16),
        in_specs=[
            pl.BlockSpec(memory_space=pltpu.VMEM),
            pl.BlockSpec(memory_space=pltpu.VMEM),
        ],
        out_specs=pl.BlockSpec(memory_space=pltpu.VMEM),
        scratch_shapes=[
            pltpu.VMEM((2 * b, c), jnp.float32),
            pltpu.VMEM((N_DEV - 1, 2 * b, c), jnp.float32),
            pltpu.VMEM((2 * b, c), jnp.float32),
            pltpu.SemaphoreType.DMA((N_DEV - 1,)),
            pltpu.SemaphoreType.DMA((N_DEV - 1,)),
            pltpu.SemaphoreType.REGULAR,
        ],
        compiler_params=pltpu.CompilerParams(collective_id=0),
    )(xt, Wp)
